# Initial kernel scaffold; baseline (speedup 1.0000x reference)
import jax
import jax.numpy as jnp
from jax.experimental import pallas as pl


def kernel(node_features, edge_features, edge_index, w_weight, w_bias, edgew_weight, edgew_bias, attn_weight):
    raise NotImplementedError("write your pallas kernel here")


# speedup vs baseline: 6.1326x; 6.1326x over previous
import functools

import jax
import jax.numpy as jnp
from jax import lax
from jax.experimental import pallas as pl
from jax.experimental.pallas import tpu as pltpu
from jax.experimental.pallas import tpu_sc as plsc

N_NODES = 10000
N_PAD = 10240
E = 320000
D = 128
DH = 64
ROWS_PER_TILE = 640
EDGES_PER_TILE = 20000
CHUNK = 80
NCHUNK = EDGES_PER_TILE // CHUNK
SLOPE = 0.2


def _tc_node(node_pad, wT, wb8, a12p):
    def body(node_ref, wT_ref, wb_ref, a12_ref, h3_ref, s12_ref):
        h = jnp.dot(node_ref[...], wT_ref[...], preferred_element_type=jnp.float32)
        h = h + wb_ref[0:1, :]
        h3_ref[0] = h[:, :DH]
        h3_ref[1] = h[:, DH:]
        s12_ref[...] = lax.dot_general(
            a12_ref[...], h, (((1,), (1,)), ((), ())),
            preferred_element_type=jnp.float32)

    blk = 512
    return pl.pallas_call(
        body,
        grid=(N_PAD // blk,),
        in_specs=[
            pl.BlockSpec((blk, D), lambda i: (i, 0)),
            pl.BlockSpec((D, D), lambda i: (0, 0)),
            pl.BlockSpec((8, D), lambda i: (0, 0)),
            pl.BlockSpec((8, D), lambda i: (0, 0)),
        ],
        out_specs=[
            pl.BlockSpec((2, blk, DH), lambda i: (0, i, 0)),
            pl.BlockSpec((8, blk), lambda i: (0, i)),
        ],
        out_shape=[
            jax.ShapeDtypeStruct((2, N_PAD, DH), jnp.float32),
            jax.ShapeDtypeStruct((8, N_PAD), jnp.float32),
        ],
    )(node_pad, wT, wb8, a12p)


def _tc_edge(edge_features, ewT, eb8, a3p):
    def body(ef_ref, ewT_ref, eb_ref, a3_ref, eh_ref, e3_ref):
        eh = jnp.dot(ef_ref[...], ewT_ref[...], preferred_element_type=jnp.float32)
        eh = eh + eb_ref[0:1, :]
        eh_ref[...] = eh
        e3_ref[...] = lax.dot_general(
            a3_ref[...], eh, (((1,), (1,)), ((), ())),
            preferred_element_type=jnp.float32)

    blk = 640
    return pl.pallas_call(
        body,
        grid=(E // blk,),
        in_specs=[
            pl.BlockSpec((blk, 16), lambda i: (i, 0)),
            pl.BlockSpec((16, D), lambda i: (0, 0)),
            pl.BlockSpec((8, D), lambda i: (0, 0)),
            pl.BlockSpec((8, D), lambda i: (0, 0)),
        ],
        out_specs=[
            pl.BlockSpec((blk, D), lambda i: (i, 0)),
            pl.BlockSpec((8, blk), lambda i: (0, i)),
        ],
        out_shape=[
            jax.ShapeDtypeStruct((E, D), jnp.float32),
            jax.ShapeDtypeStruct((8, E), jnp.float32),
        ],
    )(edge_features, ewT, eb8, a3p)


def _sc_attention(h2, s1, s2, e3, tgt, nbr):
    mesh = plsc.VectorSubcoreMesh(core_axis_name="c", subcore_axis_name="s")

    @functools.partial(
        pl.kernel,
        mesh=mesh,
        compiler_params=pltpu.CompilerParams(
            needs_layout_passes=False, use_tc_tiling_on_sc=False),
        out_type=jax.ShapeDtypeStruct((N_PAD, 2, DH), jnp.float32),
        scratch_types=[
            pltpu.VMEM_SHARED((N_PAD, DH), jnp.float32),
            pltpu.VMEM_SHARED((N_PAD,), jnp.float32),
            pltpu.VMEM((N_PAD,), jnp.float32),
            pltpu.VMEM((N_PAD,), jnp.float32),
            pltpu.VMEM((CHUNK,), jnp.int32),
            pltpu.VMEM((CHUNK,), jnp.int32),
            pltpu.VMEM((CHUNK,), jnp.int32),
            pltpu.VMEM((CHUNK,), jnp.float32),
            pltpu.VMEM((CHUNK, DH), jnp.float32),
            pltpu.VMEM((ROWS_PER_TILE,), jnp.float32),
            pltpu.VMEM((ROWS_PER_TILE,), jnp.float32),
            pltpu.VMEM((128, DH), jnp.float32),
            pltpu.VMEM((128, DH), jnp.float32),
            pltpu.VMEM((128, 1, DH), jnp.float32),
            pltpu.SemaphoreType.DMA,
        ],
    )
    def sc(h2_hbm, s1_hbm, s2_hbm, e3_hbm, tgt_hbm, nbr_hbm, out3_hbm,
           acc_sh, seg_sh, s1v, s2v, tgtb, nbrb, idxb, pb, rowsb,
           segb, recb, accb, hb, ob, dsem):
        c = lax.axis_index("c")
        s = lax.axis_index("s")
        row0 = s * ROWS_PER_TILE
        hoff = c * N_PAD

        zv = jnp.zeros((16,), jnp.float32)
        for r in range(128):
            for f in range(DH // 16):
                accb[r, pl.ds(f * 16, 16)] = zv
        for i in range(ROWS_PER_TILE // 16):
            segb[pl.ds(i * 16, 16)] = zv
        for k in range(ROWS_PER_TILE // 128):
            pltpu.sync_copy(accb, acc_sh.at[pl.ds(row0 + k * 128, 128)])
        pltpu.sync_copy(segb, seg_sh.at[pl.ds(row0, ROWS_PER_TILE)])

        pltpu.sync_copy(s1_hbm, s1v)
        pltpu.sync_copy(s2_hbm, s2v)
        plsc.subcore_barrier()

        def chunk_body(j, carry):
            base = s * EDGES_PER_TILE + j * CHUNK
            pltpu.sync_copy(tgt_hbm.at[pl.ds(base, CHUNK)], tgtb)
            pltpu.sync_copy(nbr_hbm.at[pl.ds(base, CHUNK)], nbrb)
            pltpu.sync_copy(e3_hbm.at[pl.ds(base, CHUNK)], pb)
            for i in range(CHUNK // 16):
                sl = pl.ds(i * 16, 16)
                t16 = tgtb[sl]
                n16 = nbrb[sl]
                a = plsc.load_gather(s1v, [t16]) + plsc.load_gather(s2v, [n16]) + pb[sl]
                x = jnp.where(a >= 0.0, a, a * SLOPE)
                pb[sl] = jnp.exp(x)
                idxb[sl] = n16 + hoff
            pltpu.async_copy(h2_hbm.at[idxb], rowsb, dsem).wait()
            for i in range(CHUNK // 16):
                pv16 = pb[pl.ds(i * 16, 16)]
                for j in range(16):
                    e = i * 16 + j
                    pv = jnp.full((16,), pv16[j], jnp.float32)
                    for f in range(DH // 16):
                        sl = pl.ds(f * 16, 16)
                        rowsb[e, sl] = rowsb[e, sl] * pv
            pltpu.sync_copy(rowsb, acc_sh.at[tgtb], add=True)
            pltpu.sync_copy(pb, seg_sh.at[tgtb], add=True)
            return carry

        lax.fori_loop(0, NCHUNK, chunk_body, 0)
        plsc.subcore_barrier()

        pltpu.sync_copy(seg_sh.at[pl.ds(row0, ROWS_PER_TILE)], segb)
        for i in range(ROWS_PER_TILE // 16):
            sl = pl.ds(i * 16, 16)
            recb[sl] = 1.0 / (segb[sl] + 1e-16)
        for k in range(ROWS_PER_TILE // 128):
            r0 = row0 + k * 128
            pltpu.sync_copy(acc_sh.at[pl.ds(r0, 128)], accb)
            pltpu.sync_copy(h2_hbm.at[pl.ds(hoff + r0, 128)], hb)

            def norm_body(m, carry):
                rv16 = recb[pl.ds(k * 128 + m * 16, 16)]
                for j in range(16):
                    r = m * 16 + j
                    rv = jnp.full((16,), rv16[j], jnp.float32)
                    for f in range(DH // 16):
                        sl = pl.ds(f * 16, 16)
                        ob[r, 0, sl] = accb[r, sl] * rv + hb[r, sl]
                return carry

            lax.fori_loop(0, 8, norm_body, 0)
            pltpu.sync_copy(ob, out3_hbm.at[pl.ds(r0, 128), pl.ds(c, 1)])

    return sc(h2, s1, s2, e3, tgt, nbr)


def kernel(node_features, edge_features, edge_index, w_weight, w_bias,
           edgew_weight, edgew_bias, attn_weight):
    f32 = jnp.float32
    wT = w_weight.T
    ewT = edgew_weight.T
    a12p = jnp.zeros((8, D), f32).at[0].set(attn_weight[0, :D]).at[1].set(attn_weight[0, D:2 * D])
    a3p = jnp.zeros((8, D), f32).at[0].set(attn_weight[0, 2 * D:])
    wb8 = jnp.zeros((8, D), f32).at[0].set(w_bias)
    eb8 = jnp.zeros((8, D), f32).at[0].set(edgew_bias)
    node_pad = jnp.pad(node_features, ((0, N_PAD - N_NODES), (0, 0)))

    h3, s12 = _tc_node(node_pad, wT, wb8, a12p)
    edge_h, e3a = _tc_edge(edge_features, ewT, eb8, a3p)

    h2 = h3.reshape(2 * N_PAD, DH)
    tgt = edge_index[0].astype(jnp.int32)
    nbr = edge_index[1].astype(jnp.int32)

    out3 = _sc_attention(h2, s12[0], s12[1], e3a[0], tgt, nbr)
    out = out3.reshape(N_PAD, D)[:N_NODES]
    return out, edge_h


# speedup vs baseline: 9.3563x; 1.5257x over previous
import functools

import jax
import jax.numpy as jnp
from jax import lax
from jax.experimental import pallas as pl
from jax.experimental.pallas import tpu as pltpu
from jax.experimental.pallas import tpu_sc as plsc

N_NODES = 10000
N_PAD = 10240
E = 320000
D = 128
DH = 64
ROWS_PER_TILE = 640
EDGES_PER_TILE = 20000
CHUNK = 80
NCHUNK = EDGES_PER_TILE // CHUNK
SLOPE = 0.2


def _tc_node(node_pad, wT, wb8, a12p):
    def body(node_ref, wT_ref, wb_ref, a12_ref, h3_ref, s12_ref):
        h = jnp.dot(node_ref[...], wT_ref[...], preferred_element_type=jnp.float32)
        h = h + wb_ref[0:1, :]
        h3_ref[0] = h[:, :DH]
        h3_ref[1] = h[:, DH:]
        s12_ref[...] = lax.dot_general(
            a12_ref[...], h, (((1,), (1,)), ((), ())),
            preferred_element_type=jnp.float32)

    blk = 512
    return pl.pallas_call(
        body,
        grid=(N_PAD // blk,),
        in_specs=[
            pl.BlockSpec((blk, D), lambda i: (i, 0)),
            pl.BlockSpec((D, D), lambda i: (0, 0)),
            pl.BlockSpec((8, D), lambda i: (0, 0)),
            pl.BlockSpec((8, D), lambda i: (0, 0)),
        ],
        out_specs=[
            pl.BlockSpec((2, blk, DH), lambda i: (0, i, 0)),
            pl.BlockSpec((8, blk), lambda i: (0, i)),
        ],
        out_shape=[
            jax.ShapeDtypeStruct((2, N_PAD, DH), jnp.float32),
            jax.ShapeDtypeStruct((8, N_PAD), jnp.float32),
        ],
    )(node_pad, wT, wb8, a12p)


def _tc_edge(edge_features, ewT, eb8, a3p):
    def body(ef_ref, ewT_ref, eb_ref, a3_ref, eh_ref, e3_ref):
        eh = jnp.dot(ef_ref[...], ewT_ref[...], preferred_element_type=jnp.float32)
        eh = eh + eb_ref[0:1, :]
        eh_ref[...] = eh
        e3_ref[...] = lax.dot_general(
            a3_ref[...], eh, (((1,), (1,)), ((), ())),
            preferred_element_type=jnp.float32)

    blk = 640
    return pl.pallas_call(
        body,
        grid=(E // blk,),
        in_specs=[
            pl.BlockSpec((blk, 16), lambda i: (i, 0)),
            pl.BlockSpec((16, D), lambda i: (0, 0)),
            pl.BlockSpec((8, D), lambda i: (0, 0)),
            pl.BlockSpec((8, D), lambda i: (0, 0)),
        ],
        out_specs=[
            pl.BlockSpec((blk, D), lambda i: (i, 0)),
            pl.BlockSpec((8, blk), lambda i: (0, i)),
        ],
        out_shape=[
            jax.ShapeDtypeStruct((E, D), jnp.float32),
            jax.ShapeDtypeStruct((8, E), jnp.float32),
        ],
    )(edge_features, ewT, eb8, a3p)


def _sc_attention(h2, s1, s2, e3c, tgtc, nbrc):
    mesh = plsc.VectorSubcoreMesh(core_axis_name="c", subcore_axis_name="s")

    @functools.partial(
        pl.kernel,
        mesh=mesh,
        compiler_params=pltpu.CompilerParams(
            needs_layout_passes=False, use_tc_tiling_on_sc=False),
        out_type=jax.ShapeDtypeStruct((N_PAD, 2, DH), jnp.float32),
        scratch_types=[
            pltpu.VMEM_SHARED((N_PAD, DH), jnp.float32),
            pltpu.VMEM_SHARED((N_PAD,), jnp.float32),
            pltpu.VMEM((N_PAD,), jnp.float32),
            pltpu.VMEM((N_PAD,), jnp.float32),
            pltpu.VMEM((NCHUNK, CHUNK), jnp.int32),
            [pltpu.VMEM((CHUNK,), jnp.int32)] * 2,
            [pltpu.VMEM((CHUNK,), jnp.float32)] * 2,
            [pltpu.VMEM((CHUNK,), jnp.int32)] * 2,
            [pltpu.VMEM((CHUNK,), jnp.float32)] * 2,
            [pltpu.VMEM((CHUNK, DH), jnp.float32)] * 2,
            pltpu.VMEM((ROWS_PER_TILE,), jnp.float32),
            pltpu.VMEM((ROWS_PER_TILE,), jnp.float32),
            pltpu.VMEM((128, DH), jnp.float32),
            pltpu.VMEM((128, DH), jnp.float32),
            pltpu.VMEM((128, 1, DH), jnp.float32),
            [pltpu.SemaphoreType.DMA] * 2,
            [pltpu.SemaphoreType.DMA] * 2,
            [pltpu.SemaphoreType.DMA] * 2,
            [pltpu.SemaphoreType.DMA] * 2,
            pltpu.SemaphoreType.DMA,
        ],
    )
    def sc(h2_hbm, s1_hbm, s2_hbm, e3_hbm, tgt_hbm, nbr_hbm, out3_hbm,
           acc_sh, seg_sh, s1v, s2v, tgta, nbrb, e3b, idxb, pb, rowsb,
           segb, recb, accb, hb, ob, gsem, ssem, psem, isem, dsem):
        c = lax.axis_index("c")
        s = lax.axis_index("s")
        row0 = s * ROWS_PER_TILE
        hoff = c * N_PAD
        chunk0 = s * NCHUNK

        cp0 = pltpu.async_copy(tgt_hbm.at[pl.ds(chunk0, NCHUNK)], tgta, dsem)
        cp3 = pltpu.async_copy(s1_hbm, s1v, dsem)
        cp4 = pltpu.async_copy(s2_hbm, s2v, dsem)

        zv = jnp.zeros((16,), jnp.float32)
        for r in range(128):
            for f in range(DH // 16):
                accb[r, pl.ds(f * 16, 16)] = zv
        for i in range(ROWS_PER_TILE // 16):
            segb[pl.ds(i * 16, 16)] = zv
        for k in range(ROWS_PER_TILE // 128):
            pltpu.sync_copy(accb, acc_sh.at[pl.ds(row0 + k * 128, 128)])
        pltpu.sync_copy(segb, seg_sh.at[pl.ds(row0, ROWS_PER_TILE)])
        cp0.wait(); cp3.wait(); cp4.wait()
        plsc.subcore_barrier()

        def issue_idx_loads(jj, bb):
            pltpu.async_copy(nbr_hbm.at[chunk0 + jj], nbrb[bb], isem[bb])
            pltpu.async_copy(e3_hbm.at[chunk0 + jj], e3b[bb], isem[bb])

        def wait_idx_loads(jj, bb):
            pltpu.make_async_copy(nbr_hbm.at[chunk0 + jj], nbrb[bb], isem[bb]).wait()
            pltpu.make_async_copy(e3_hbm.at[chunk0 + jj], e3b[bb], isem[bb]).wait()

        def compute_chunk(jj, bb):
            for i in range(CHUNK // 16):
                sl = pl.ds(i * 16, 16)
                t16 = tgta[jj, sl]
                n16 = nbrb[bb][sl]
                a = (plsc.load_gather(s1v, [t16]) + plsc.load_gather(s2v, [n16])
                     + e3b[bb][sl])
                x = jnp.where(a >= 0.0, a, a * SLOPE)
                pb[bb][sl] = jnp.exp(x)
                idxb[bb][sl] = n16 + hoff

        def issue_gather(bb):
            return pltpu.async_copy(h2_hbm.at[idxb[bb]], rowsb[bb], gsem[bb])

        issue_idx_loads(0, 0)
        wait_idx_loads(0, 0)
        compute_chunk(0, 0)
        issue_gather(0)
        issue_idx_loads(1, 1)

        def pair_body(j0, carry):
            for b in range(2):
                j = 2 * j0 + b
                o = 1 - b
                pltpu.make_async_copy(h2_hbm.at[idxb[b]], rowsb[b], gsem[b]).wait()
                @pl.when(j >= 1)
                def _():
                    pltpu.make_async_copy(
                        rowsb[o], acc_sh.at[tgta.at[j - 1]], ssem[o]).wait()
                    pltpu.make_async_copy(
                        pb[o], seg_sh.at[tgta.at[j - 1]], psem[o]).wait()
                @pl.when(j + 1 < NCHUNK)
                def _():
                    wait_idx_loads(j + 1, o)
                    compute_chunk(j + 1, o)
                    issue_gather(o)
                @pl.when(j + 2 < NCHUNK)
                def _():
                    issue_idx_loads(j + 2, b)
                for i in range(CHUNK // 16):
                    pv16 = pb[b][pl.ds(i * 16, 16)]
                    for jj in range(16):
                        e = i * 16 + jj
                        pv = jnp.full((16,), pv16[jj], jnp.float32)
                        for f in range(DH // 16):
                            sl = pl.ds(f * 16, 16)
                            rowsb[b][e, sl] = rowsb[b][e, sl] * pv
                pltpu.async_copy(rowsb[b], acc_sh.at[tgta.at[j]], ssem[b], add=True)
                pltpu.async_copy(pb[b], seg_sh.at[tgta.at[j]], psem[b], add=True)
            return carry

        lax.fori_loop(0, NCHUNK // 2, pair_body, 0)
        pltpu.make_async_copy(
            rowsb[1], acc_sh.at[tgta.at[NCHUNK - 1]], ssem[1]).wait()
        pltpu.make_async_copy(
            pb[1], seg_sh.at[tgta.at[NCHUNK - 1]], psem[1]).wait()
        plsc.subcore_barrier()

        pltpu.sync_copy(seg_sh.at[pl.ds(row0, ROWS_PER_TILE)], segb)
        for i in range(ROWS_PER_TILE // 16):
            sl = pl.ds(i * 16, 16)
            recb[sl] = 1.0 / (segb[sl] + 1e-16)
        for k in range(ROWS_PER_TILE // 128):
            r0 = row0 + k * 128
            pltpu.sync_copy(acc_sh.at[pl.ds(r0, 128)], accb)
            pltpu.sync_copy(h2_hbm.at[pl.ds(hoff + r0, 128)], hb)

            def norm_body(m, carry):
                rv16 = recb[pl.ds(k * 128 + m * 16, 16)]
                for j in range(16):
                    r = m * 16 + j
                    rv = jnp.full((16,), rv16[j], jnp.float32)
                    for f in range(DH // 16):
                        sl = pl.ds(f * 16, 16)
                        ob[r, 0, sl] = accb[r, sl] * rv + hb[r, sl]
                return carry

            lax.fori_loop(0, 8, norm_body, 0)
            pltpu.sync_copy(ob, out3_hbm.at[pl.ds(r0, 128), pl.ds(c, 1)])

    return sc(h2, s1, s2, e3c, tgtc, nbrc)


def kernel(node_features, edge_features, edge_index, w_weight, w_bias,
           edgew_weight, edgew_bias, attn_weight):
    f32 = jnp.float32
    wT = w_weight.T
    ewT = edgew_weight.T
    a12p = jnp.zeros((8, D), f32).at[0].set(attn_weight[0, :D]).at[1].set(attn_weight[0, D:2 * D])
    a3p = jnp.zeros((8, D), f32).at[0].set(attn_weight[0, 2 * D:])
    wb8 = jnp.zeros((8, D), f32).at[0].set(w_bias)
    eb8 = jnp.zeros((8, D), f32).at[0].set(edgew_bias)
    node_pad = jnp.pad(node_features, ((0, N_PAD - N_NODES), (0, 0)))

    h3, s12 = _tc_node(node_pad, wT, wb8, a12p)
    edge_h, e3a = _tc_edge(edge_features, ewT, eb8, a3p)

    h2 = h3.reshape(2 * N_PAD, DH)
    tgt = edge_index[0].astype(jnp.int32).reshape(E // CHUNK, CHUNK)
    nbr = edge_index[1].astype(jnp.int32).reshape(E // CHUNK, CHUNK)
    e3 = e3a[0].reshape(E // CHUNK, CHUNK)

    out3 = _sc_attention(h2, s12[0], s12[1], e3, tgt, nbr)
    out = out3.reshape(N_PAD, D)[:N_NODES]
    return out, edge_h


# speedup vs baseline: 14.3519x; 1.5339x over previous
import functools

import jax
import jax.numpy as jnp
from jax import lax
from jax.experimental import pallas as pl
from jax.experimental.pallas import tpu as pltpu
from jax.experimental.pallas import tpu_sc as plsc

N_NODES = 10000
N_PAD = 10240
E = 320000
D = 128
DH = 64
ROWS_PER_TILE = 640
EDGES_PER_TILE = 20000
CHUNK = 80
NCHUNK = EDGES_PER_TILE // CHUNK
SLOPE = 0.2


def _tc_node(node_pad, wT, wb8, a12p, c38):
    def body(node_ref, wT_ref, wb_ref, a12_ref, c3_ref, h3_ref, s12_ref):
        h = jnp.dot(node_ref[...], wT_ref[...], preferred_element_type=jnp.float32)
        h = h + wb_ref[0:1, :]
        h3_ref[0] = h[:, :DH]
        h3_ref[1] = h[:, DH:]
        s12_ref[...] = lax.dot_general(
            a12_ref[...], h, (((1,), (1,)), ((), ())),
            preferred_element_type=jnp.float32) + c3_ref[:, 0:1]

    blk = 512
    return pl.pallas_call(
        body,
        grid=(N_PAD // blk,),
        in_specs=[
            pl.BlockSpec((blk, D), lambda i: (i, 0)),
            pl.BlockSpec((D, D), lambda i: (0, 0)),
            pl.BlockSpec((8, D), lambda i: (0, 0)),
            pl.BlockSpec((8, D), lambda i: (0, 0)),
            pl.BlockSpec((8, 8), lambda i: (0, 0)),
        ],
        out_specs=[
            pl.BlockSpec((2, blk, DH), lambda i: (0, i, 0)),
            pl.BlockSpec((8, blk), lambda i: (0, i)),
        ],
        out_shape=[
            jax.ShapeDtypeStruct((2, N_PAD, DH), jnp.float32),
            jax.ShapeDtypeStruct((8, N_PAD), jnp.float32),
        ],
    )(node_pad, wT, wb8, a12p, c38)


def _tc_edge(edge_features, ewT, eb8):
    def body(ef_ref, ewT_ref, eb_ref, eh_ref):
        eh = jnp.dot(ef_ref[...], ewT_ref[...], preferred_element_type=jnp.float32)
        eh_ref[...] = eh + eb_ref[0:1, :]

    blk = 2560
    return pl.pallas_call(
        body,
        grid=(E // blk,),
        in_specs=[
            pl.BlockSpec((blk, 16), lambda i: (i, 0)),
            pl.BlockSpec((16, D), lambda i: (0, 0)),
            pl.BlockSpec((8, D), lambda i: (0, 0)),
        ],
        out_specs=pl.BlockSpec((blk, D), lambda i: (i, 0)),
        out_shape=jax.ShapeDtypeStruct((E, D), jnp.float32),
    )(edge_features, ewT, eb8)


def _tc_e3(edge_features, va38):
    def body(ef_ref, va3_ref, e3_ref):
        e3_ref[...] = lax.dot_general(
            va3_ref[...], ef_ref[...], (((1,), (1,)), ((), ())),
            preferred_element_type=jnp.float32)

    blk = 2560
    return pl.pallas_call(
        body,
        grid=(E // blk,),
        in_specs=[
            pl.BlockSpec((blk, 16), lambda i: (i, 0)),
            pl.BlockSpec((8, 16), lambda i: (0, 0)),
        ],
        out_specs=pl.BlockSpec((8, blk), lambda i: (0, i)),
        out_shape=jax.ShapeDtypeStruct((8, E), jnp.float32),
    )(edge_features, va38)


def _sc_attention(h2, s1, s2, e3c, tgtc, nbrc):
    mesh = plsc.VectorSubcoreMesh(core_axis_name="c", subcore_axis_name="s")

    @functools.partial(
        pl.kernel,
        mesh=mesh,
        compiler_params=pltpu.CompilerParams(
            needs_layout_passes=False, use_tc_tiling_on_sc=False),
        out_type=jax.ShapeDtypeStruct((N_PAD, 2, DH), jnp.float32),
        scratch_types=[
            pltpu.VMEM_SHARED((N_PAD, DH), jnp.float32),
            pltpu.VMEM_SHARED((N_PAD,), jnp.float32),
            pltpu.VMEM((N_PAD,), jnp.float32),
            pltpu.VMEM((N_PAD,), jnp.float32),
            pltpu.VMEM((NCHUNK, CHUNK), jnp.int32),
            [pltpu.VMEM((CHUNK,), jnp.int32)] * 2,
            [pltpu.VMEM((CHUNK,), jnp.float32)] * 2,
            [pltpu.VMEM((CHUNK,), jnp.int32)] * 2,
            [pltpu.VMEM((CHUNK,), jnp.float32)] * 2,
            [pltpu.VMEM((CHUNK, DH), jnp.float32)] * 2,
            [pltpu.VMEM((CHUNK, DH), jnp.float32)] * 2,
            pltpu.VMEM((ROWS_PER_TILE,), jnp.float32),
            pltpu.VMEM((ROWS_PER_TILE,), jnp.float32),
            pltpu.VMEM((128, DH), jnp.float32),
            pltpu.VMEM((128, DH), jnp.float32),
            pltpu.VMEM((128, 1, DH), jnp.float32),
            [pltpu.SemaphoreType.DMA] * 2,
            [pltpu.SemaphoreType.DMA] * 2,
            [pltpu.SemaphoreType.DMA] * 2,
            [pltpu.SemaphoreType.DMA] * 2,
            pltpu.SemaphoreType.DMA,
        ],
    )
    def sc(h2_hbm, s1_hbm, s2_hbm, e3_hbm, tgt_hbm, nbr_hbm, out3_hbm,
           acc_sh, seg_sh, s1v, s2v, tgta, nbrb, e3b, idxb, pb, rowsb, sbuf,
           segb, recb, accb, hb, ob, gsem, ssem, psem, isem, dsem):
        c = lax.axis_index("c")
        s = lax.axis_index("s")
        row0 = s * ROWS_PER_TILE
        hoff = c * N_PAD
        chunk0 = s * NCHUNK

        cp0 = pltpu.async_copy(tgt_hbm.at[pl.ds(chunk0, NCHUNK)], tgta, dsem)
        cp3 = pltpu.async_copy(s1_hbm, s1v, dsem)
        cp4 = pltpu.async_copy(s2_hbm, s2v, dsem)

        zv = jnp.zeros((16,), jnp.float32)
        for r in range(128):
            for f in range(DH // 16):
                accb[r, pl.ds(f * 16, 16)] = zv
        for i in range(ROWS_PER_TILE // 16):
            segb[pl.ds(i * 16, 16)] = zv
        for k in range(ROWS_PER_TILE // 128):
            pltpu.sync_copy(accb, acc_sh.at[pl.ds(row0 + k * 128, 128)])
        pltpu.sync_copy(segb, seg_sh.at[pl.ds(row0, ROWS_PER_TILE)])
        cp0.wait(); cp3.wait(); cp4.wait()
        plsc.subcore_barrier()

        def issue_idx_loads(jj, bb):
            pltpu.async_copy(nbr_hbm.at[chunk0 + jj], nbrb[bb], isem[bb])
            pltpu.async_copy(e3_hbm.at[chunk0 + jj], e3b[bb], isem[bb])

        def wait_idx_loads(jj, bb):
            pltpu.make_async_copy(nbr_hbm.at[chunk0 + jj], nbrb[bb], isem[bb]).wait()
            pltpu.make_async_copy(e3_hbm.at[chunk0 + jj], e3b[bb], isem[bb]).wait()

        def compute_chunk(jj, bb):
            for i in range(CHUNK // 16):
                sl = pl.ds(i * 16, 16)
                t16 = tgta[jj, sl]
                n16 = nbrb[bb][sl]
                a = (plsc.load_gather(s1v, [t16]) + plsc.load_gather(s2v, [n16])
                     + e3b[bb][sl])
                x = jnp.where(a >= 0.0, a, a * SLOPE)
                pb[bb][sl] = jnp.exp(x)
                idxb[bb][sl] = n16 + hoff

        def issue_gather(bb):
            return pltpu.async_copy(h2_hbm.at[idxb[bb]], rowsb[bb], gsem[bb])

        issue_idx_loads(0, 0)
        wait_idx_loads(0, 0)
        compute_chunk(0, 0)
        issue_gather(0)
        issue_idx_loads(1, 1)

        def pair_body(j0, carry):
            for b in range(2):
                j = 2 * j0 + b
                o = 1 - b
                @pl.when(j >= 1)
                def _():
                    pltpu.make_async_copy(
                        pb[o], seg_sh.at[tgta.at[j - 1]], psem[o]).wait()
                @pl.when(j + 1 < NCHUNK)
                def _():
                    wait_idx_loads(j + 1, o)
                    compute_chunk(j + 1, o)
                    issue_gather(o)
                @pl.when(j + 2 < NCHUNK)
                def _():
                    issue_idx_loads(j + 2, b)
                pltpu.make_async_copy(h2_hbm.at[idxb[b]], rowsb[b], gsem[b]).wait()
                @pl.when(j >= 2)
                def _():
                    pltpu.make_async_copy(
                        sbuf[b], acc_sh.at[tgta.at[j - 2]], ssem[b]).wait()
                for i in range(CHUNK // 16):
                    pv16 = pb[b][pl.ds(i * 16, 16)]
                    for jj in range(16):
                        e = i * 16 + jj
                        pv = jnp.full((16,), pv16[jj], jnp.float32)
                        for f in range(DH // 16):
                            sl = pl.ds(f * 16, 16)
                            sbuf[b][e, sl] = rowsb[b][e, sl] * pv
                pltpu.async_copy(sbuf[b], acc_sh.at[tgta.at[j]], ssem[b], add=True)
                pltpu.async_copy(pb[b], seg_sh.at[tgta.at[j]], psem[b], add=True)
            return carry

        lax.fori_loop(0, NCHUNK // 2, pair_body, 0)
        pltpu.make_async_copy(
            sbuf[0], acc_sh.at[tgta.at[NCHUNK - 2]], ssem[0]).wait()
        pltpu.make_async_copy(
            sbuf[1], acc_sh.at[tgta.at[NCHUNK - 1]], ssem[1]).wait()
        pltpu.make_async_copy(
            pb[1], seg_sh.at[tgta.at[NCHUNK - 1]], psem[1]).wait()
        plsc.subcore_barrier()

        pltpu.sync_copy(seg_sh.at[pl.ds(row0, ROWS_PER_TILE)], segb)
        for i in range(ROWS_PER_TILE // 16):
            sl = pl.ds(i * 16, 16)
            recb[sl] = 1.0 / (segb[sl] + 1e-16)
        for k in range(ROWS_PER_TILE // 128):
            r0 = row0 + k * 128
            pltpu.sync_copy(acc_sh.at[pl.ds(r0, 128)], accb)
            pltpu.sync_copy(h2_hbm.at[pl.ds(hoff + r0, 128)], hb)

            def norm_body(m, carry):
                rv16 = recb[pl.ds(k * 128 + m * 16, 16)]
                for j in range(16):
                    r = m * 16 + j
                    rv = jnp.full((16,), rv16[j], jnp.float32)
                    for f in range(DH // 16):
                        sl = pl.ds(f * 16, 16)
                        ob[r, 0, sl] = accb[r, sl] * rv + hb[r, sl]
                return carry

            lax.fori_loop(0, 8, norm_body, 0)
            pltpu.sync_copy(ob, out3_hbm.at[pl.ds(r0, 128), pl.ds(c, 1)])

    return sc(h2, s1, s2, e3c, tgtc, nbrc)


def kernel(node_features, edge_features, edge_index, w_weight, w_bias,
           edgew_weight, edgew_bias, attn_weight):
    f32 = jnp.float32
    wT = w_weight.T
    ewT = edgew_weight.T
    a3 = attn_weight[0, 2 * D:]
    a12p = jnp.zeros((8, D), f32).at[0].set(attn_weight[0, :D]).at[1].set(attn_weight[0, D:2 * D])
    wb8 = jnp.zeros((8, D), f32).at[0].set(w_bias)
    eb8 = jnp.zeros((8, D), f32).at[0].set(edgew_bias)
    va38 = jnp.zeros((8, 16), f32).at[0].set(a3 @ edgew_weight)
    c38 = jnp.zeros((8, 8), f32).at[0].set(jnp.dot(edgew_bias, a3))
    node_pad = jnp.pad(node_features, ((0, N_PAD - N_NODES), (0, 0)))

    h3, s12 = _tc_node(node_pad, wT, wb8, a12p, c38)
    e3a = _tc_e3(edge_features, va38)
    edge_h = _tc_edge(edge_features, ewT, eb8)

    h2 = h3.reshape(2 * N_PAD, DH)
    tgt = edge_index[0].astype(jnp.int32).reshape(E // CHUNK, CHUNK)
    nbr = edge_index[1].astype(jnp.int32).reshape(E // CHUNK, CHUNK)
    e3 = e3a[0].reshape(E // CHUNK, CHUNK)

    out3 = _sc_attention(h2, s12[0], s12[1], e3, tgt, nbr)
    out = out3.reshape(N_PAD, D)[:N_NODES]
    return out, edge_h
